# baseline (speedup 1.0000x reference)
import functools

import jax
import jax.numpy as jnp
from jax import lax
from jax.experimental import pallas as pl
from jax.experimental.pallas import tpu as pltpu
from jax.experimental.pallas import tpu_sc as plsc

N = 10000
E = 320000
D_IN = 128
D_H = 16
D_OUT = 64

NC = 2
NS = 16
NW = NC * NS
EPT = E // NW
CB = 80
NCHUNK = EPT // CB
NBUF = 5
NGRP = NCHUNK // NBUF
ACC_N = 10240
RPS = ACC_N // NS
LASTR = N - (NS - 1) * RPS
ZR = 128

_MESH = plsc.VectorSubcoreMesh(
    core_axis_name="c", subcore_axis_name="s", num_cores=NC, num_subcores=NS)


def _make_deg():
  scratch = [
      pltpu.VMEM((NCHUNK, CB), jnp.int32),
      pltpu.VMEM((ZR, D_H), jnp.float32),
      pltpu.VMEM((CB, D_H), jnp.float32),
      pltpu.VMEM_SHARED((ACC_N, D_H), jnp.float32),
  ]

  @functools.partial(
      pl.kernel,
      out_type=jax.ShapeDtypeStruct((NC, N, D_H), jnp.float32),
      mesh=_MESH,
      scratch_types=scratch,
      compiler_params=pltpu.CompilerParams(use_tc_tiling_on_sc=False),
  )
  def deg_kernel(ei_hbm, out_hbm, dst_v, zbuf, obuf, acc):
    c = lax.axis_index("c")
    s = lax.axis_index("s")
    wid = c * NS + s
    pltpu.sync_copy(ei_hbm.at[1, wid], dst_v)

    def fill_z(r, _):
      zbuf[r, :] = jnp.zeros((D_H,), jnp.float32)
      return 0
    lax.fori_loop(0, ZR, fill_z, 0)

    def fill_o(r, _):
      obuf[r, :] = jnp.ones((D_H,), jnp.float32)
      return 0
    lax.fori_loop(0, CB, fill_o, 0)

    for k in range(RPS // ZR):
      pltpu.sync_copy(zbuf, acc.at[pl.ds(s * RPS + k * ZR, ZR)])
    plsc.subcore_barrier()

    def chunk(i, _):
      pltpu.sync_copy(obuf, acc.at[dst_v.at[i]], add=True)
      return 0
    lax.fori_loop(0, NCHUNK, chunk, 0)

    plsc.subcore_barrier()
    _drain(acc, out_hbm, c, s)

  return deg_kernel


def _drain(acc, out_hbm, c, s):
  @pl.when(s < NS - 1)
  def _():
    pltpu.sync_copy(acc.at[pl.ds(s * RPS, RPS)],
                    out_hbm.at[c, pl.ds(s * RPS, RPS)])

  @pl.when(s == NS - 1)
  def _():
    pltpu.sync_copy(acc.at[pl.ds((NS - 1) * RPS, LASTR)],
                    out_hbm.at[c, pl.ds((NS - 1) * RPS, LASTR)])


def _make_agg(d):
  scratch = [
      pltpu.VMEM((NCHUNK, CB), jnp.int32),
      pltpu.VMEM((NCHUNK, CB), jnp.int32),
      [pltpu.VMEM((CB, d), jnp.float32) for _ in range(NBUF)],
      pltpu.VMEM((ZR, d), jnp.float32),
      pltpu.VMEM_SHARED((ACC_N, d), jnp.float32),
      [pltpu.SemaphoreType.DMA for _ in range(NBUF)],
  ]

  @functools.partial(
      pl.kernel,
      out_type=jax.ShapeDtypeStruct((NC, N, d), jnp.float32),
      mesh=_MESH,
      scratch_types=scratch,
      compiler_params=pltpu.CompilerParams(use_tc_tiling_on_sc=False),
  )
  def agg_kernel(y_hbm, ei_hbm, out_hbm, src_v, dst_v, bufs, zbuf, acc,
                 gsems):
    c = lax.axis_index("c")
    s = lax.axis_index("s")
    wid = c * NS + s
    pltpu.sync_copy(ei_hbm.at[0, wid], src_v)
    pltpu.sync_copy(ei_hbm.at[1, wid], dst_v)

    def fill_z(r, _):
      for j in range(d // 16):
        zbuf[r, pl.ds(j * 16, 16)] = jnp.zeros((16,), jnp.float32)
      return 0
    lax.fori_loop(0, ZR, fill_z, 0)

    for k in range(RPS // ZR):
      pltpu.sync_copy(zbuf, acc.at[pl.ds(s * RPS + k * ZR, ZR)])
    plsc.subcore_barrier()

    def gather(i, b, gs):
      pltpu.async_copy(y_hbm.at[src_v.at[i]], b, gs)

    def wait_gather(i, b, gs):
      pltpu.make_async_copy(y_hbm.at[src_v.at[i]], b, gs).wait()

    def sync_scatter(i, b):
      pltpu.sync_copy(b, acc.at[dst_v.at[i]], add=True)

    for b in range(NBUF):
      gather(b, bufs[b], gsems[b])

    def group(j, _):
      g = j * NBUF
      for b in range(NBUF):
        wait_gather(g + b, bufs[b], gsems[b])
        sync_scatter(g + b, bufs[b])
        gather(g + NBUF + b, bufs[b], gsems[b])
      return 0
    lax.fori_loop(0, NGRP - 1, group, 0)

    last = (NGRP - 1) * NBUF
    for b in range(NBUF):
      wait_gather(last + b, bufs[b], gsems[b])
      sync_scatter(last + b, bufs[b])

    plsc.subcore_barrier()
    _drain(acc, out_hbm, c, s)

  return agg_kernel


_deg_call = _make_deg()
_agg16_call = _make_agg(D_H)
_agg64_call = _make_agg(D_OUT)


BN = 2000
BP16 = BN * D_H // 128


def _layer1_body(x_ref, w1b_ref, dg_ref, y1_ref, dinv_ref):
  deg = dg_ref[0, 0] + dg_ref[1, 0] + 1.0
  dinv = lax.rsqrt(deg)
  xw = jnp.dot(x_ref[0], w1b_ref[...], preferred_element_type=jnp.float32)
  y1_ref[0] = xw * dinv
  dinv_ref[0] = dinv


def _layer1_call(x_r8, w1b, degp_p):
  return pl.pallas_call(
      _layer1_body,
      grid=(N // BN,),
      in_specs=[
          pl.BlockSpec((1, BP16, 8 * D_IN), lambda i: (i, 0, 0)),
          pl.BlockSpec((8 * D_IN, 128), lambda i: (0, 0)),
          pl.BlockSpec((NC, 1, BP16, 128), lambda i: (0, i, 0, 0)),
      ],
      out_specs=[
          pl.BlockSpec((1, BP16, 128), lambda i: (i, 0, 0)),
          pl.BlockSpec((1, BP16, 128), lambda i: (i, 0, 0)),
      ],
      out_shape=[
          jax.ShapeDtypeStruct((N // BN, BP16, 128), jnp.float32),
          jax.ShapeDtypeStruct((N // BN, BP16, 128), jnp.float32),
      ],
  )(x_r8, w1b, degp_p)


def _layer2_body(ag_ref, y1_ref, dinv_ref, b1_ref, w2b_ref, es_ref, y2_ref):
  t = ag_ref[0, 0] + ag_ref[1, 0] + y1_ref[0]
  h = jnp.maximum(dinv_ref[0] * t + b1_ref[...], 0.0)
  hw = jnp.dot(h, w2b_ref[...], preferred_element_type=jnp.float32)
  dinv64 = jnp.dot(dinv_ref[0], es_ref[...],
                   preferred_element_type=jnp.float32)
  y2_ref[0] = hw * dinv64


def _layer2_call(agg1_p, y1p, dinvp, b1t, w2b, es):
  return pl.pallas_call(
      _layer2_body,
      grid=(N // BN,),
      in_specs=[
          pl.BlockSpec((NC, 1, BP16, 128), lambda i: (0, i, 0, 0)),
          pl.BlockSpec((1, BP16, 128), lambda i: (i, 0, 0)),
          pl.BlockSpec((1, BP16, 128), lambda i: (i, 0, 0)),
          pl.BlockSpec((1, 128), lambda i: (0, 0)),
          pl.BlockSpec((128, 8 * D_OUT), lambda i: (0, 0)),
          pl.BlockSpec((128, 8 * D_OUT), lambda i: (0, 0)),
      ],
      out_specs=pl.BlockSpec((1, BP16, 8 * D_OUT), lambda i: (i, 0, 0)),
      out_shape=jax.ShapeDtypeStruct((N // BN, BP16, 8 * D_OUT), jnp.float32),
  )(agg1_p, y1p, dinvp, b1t, w2b, es)


def _out_body(ag_ref, y2_ref, dinv_ref, es_ref, b2_ref, ms_ref, o_ref):
  dinv64 = jnp.dot(dinv_ref[0], es_ref[...],
                   preferred_element_type=jnp.float32)
  o = dinv64 * (ag_ref[0, 0] + ag_ref[1, 0] + y2_ref[0]) + b2_ref[...]
  m = jnp.max(o, axis=1, keepdims=True)
  s = jnp.dot(jnp.exp(o - m), ms_ref[...], preferred_element_type=jnp.float32)
  o_ref[0] = o - (jnp.log(s) + m)


def _out_call(agg2_v, y2p, dinvp, es, b2t, ms):
  return pl.pallas_call(
      _out_body,
      grid=(N // BN,),
      in_specs=[
          pl.BlockSpec((NC, 1, BP16, 8 * D_OUT), lambda i: (0, i, 0, 0)),
          pl.BlockSpec((1, BP16, 8 * D_OUT), lambda i: (i, 0, 0)),
          pl.BlockSpec((1, BP16, 128), lambda i: (i, 0, 0)),
          pl.BlockSpec((128, 8 * D_OUT), lambda i: (0, 0)),
          pl.BlockSpec((1, 8 * D_OUT), lambda i: (0, 0)),
          pl.BlockSpec((8 * D_OUT, 8 * D_OUT), lambda i: (0, 0)),
      ],
      out_specs=pl.BlockSpec((1, BP16, 8 * D_OUT), lambda i: (i, 0, 0)),
      out_shape=jax.ShapeDtypeStruct((N // BN, BP16, 8 * D_OUT), jnp.float32),
  )(agg2_v, y2p, dinvp, es, b2t, ms)


def kernel(x, edge_index, W1, b1, W2, b2):
  f32 = jnp.float32
  eye8 = jnp.eye(8, dtype=f32)
  ei = edge_index.astype(jnp.int32).reshape(2, NW, NCHUNK, CB)

  nb = N // BN
  degp = _deg_call(ei)
  degp_p = degp.reshape(NC, nb, BP16, 128)

  x_r8 = x.reshape(nb, BP16, 8 * D_IN)
  w1b = jnp.kron(eye8, W1)
  y1p, dinvp = _layer1_call(x_r8, w1b, degp_p)

  sel = (jnp.arange(D_H) == 0).astype(f32)[:, None] * jnp.ones((1, D_OUT), f32)
  es = jnp.kron(eye8, sel)

  agg1 = _agg16_call(y1p.reshape(N, D_H), ei)
  w2b = jnp.kron(eye8, W2)
  b1t = jnp.tile(b1, 8).reshape(1, 128)
  y2p = _layer2_call(agg1.reshape(NC, nb, BP16, 128), y1p, dinvp, b1t, w2b,
                     es)

  agg2 = _agg64_call(y2p.reshape(N, D_OUT), ei)
  ms = jnp.kron(eye8, jnp.ones((D_OUT, D_OUT), f32))
  b2t = jnp.tile(b2, 8).reshape(1, 8 * D_OUT)
  outp = _out_call(agg2.reshape(NC, nb, BP16, 8 * D_OUT), y2p, dinvp, es,
                   b2t, ms)
  return outp.reshape(N, D_OUT)


# speedup vs baseline: 1.0226x; 1.0226x over previous
import functools

import jax
import jax.numpy as jnp
from jax import lax
from jax.experimental import pallas as pl
from jax.experimental.pallas import tpu as pltpu
from jax.experimental.pallas import tpu_sc as plsc

N = 10000
E = 320000
D_IN = 128
D_H = 16
D_OUT = 64

NC = 2
NS = 16
NW = NC * NS
EPT = E // NW
CB = 80
NCHUNK = EPT // CB
NBUF = 5
NGRP = NCHUNK // NBUF
ACC_N = 10240
RPS = ACC_N // NS
LASTR = N - (NS - 1) * RPS
ZR = 128

_MESH = plsc.VectorSubcoreMesh(
    core_axis_name="c", subcore_axis_name="s", num_cores=NC, num_subcores=NS)


def _make_deg():
  scratch = [
      pltpu.VMEM((NCHUNK, CB), jnp.int32),
      pltpu.VMEM((ZR, D_H), jnp.float32),
      pltpu.VMEM((CB, D_H), jnp.float32),
      pltpu.VMEM_SHARED((ACC_N, D_H), jnp.float32),
  ]

  @functools.partial(
      pl.kernel,
      out_type=jax.ShapeDtypeStruct((NC, N, D_H), jnp.float32),
      mesh=_MESH,
      scratch_types=scratch,
      compiler_params=pltpu.CompilerParams(use_tc_tiling_on_sc=False),
  )
  def deg_kernel(ei_hbm, out_hbm, dst_v, zbuf, obuf, acc):
    c = lax.axis_index("c")
    s = lax.axis_index("s")
    wid = c * NS + s
    pltpu.sync_copy(ei_hbm.at[1, wid], dst_v)

    def fill_z(r, _):
      zbuf[r, :] = jnp.zeros((D_H,), jnp.float32)
      return 0
    lax.fori_loop(0, ZR, fill_z, 0)

    def fill_o(r, _):
      obuf[r, :] = jnp.ones((D_H,), jnp.float32)
      return 0
    lax.fori_loop(0, CB, fill_o, 0)

    for k in range(RPS // ZR):
      pltpu.sync_copy(zbuf, acc.at[pl.ds(s * RPS + k * ZR, ZR)])
    plsc.subcore_barrier()

    def chunk(i, _):
      pltpu.sync_copy(obuf, acc.at[dst_v.at[i]], add=True)
      return 0
    lax.fori_loop(0, NCHUNK, chunk, 0)

    plsc.subcore_barrier()
    _drain(acc, out_hbm, c, s)

  return deg_kernel


def _drain(acc, out_hbm, c, s):
  @pl.when(s < NS - 1)
  def _():
    pltpu.sync_copy(acc.at[pl.ds(s * RPS, RPS)],
                    out_hbm.at[c, pl.ds(s * RPS, RPS)])

  @pl.when(s == NS - 1)
  def _():
    pltpu.sync_copy(acc.at[pl.ds((NS - 1) * RPS, LASTR)],
                    out_hbm.at[c, pl.ds((NS - 1) * RPS, LASTR)])


def _make_agg(d, stage_y):
  scratch = [
      pltpu.VMEM((NCHUNK, CB), jnp.int32),
      pltpu.VMEM((NCHUNK, CB), jnp.int32),
      [pltpu.VMEM((CB, d), jnp.float32) for _ in range(NBUF)],
      pltpu.VMEM((ZR, d), jnp.float32),
      pltpu.VMEM_SHARED((ACC_N, d), jnp.float32),
      pltpu.VMEM_SHARED((N if stage_y else 8, d), jnp.float32),
      [pltpu.SemaphoreType.DMA for _ in range(NBUF)],
  ]

  @functools.partial(
      pl.kernel,
      out_type=jax.ShapeDtypeStruct((NC, N, d), jnp.float32),
      mesh=_MESH,
      scratch_types=scratch,
      compiler_params=pltpu.CompilerParams(use_tc_tiling_on_sc=False),
  )
  def agg_kernel(y_hbm, ei_hbm, out_hbm, src_v, dst_v, bufs, zbuf, acc,
                 y_sh, gsems):
    c = lax.axis_index("c")
    s = lax.axis_index("s")
    wid = c * NS + s
    pltpu.sync_copy(ei_hbm.at[0, wid], src_v)
    pltpu.sync_copy(ei_hbm.at[1, wid], dst_v)

    if stage_y:
      @pl.when(s < NS - 1)
      def _():
        pltpu.sync_copy(y_hbm.at[pl.ds(s * RPS, RPS)],
                        y_sh.at[pl.ds(s * RPS, RPS)])

      @pl.when(s == NS - 1)
      def _():
        pltpu.sync_copy(y_hbm.at[pl.ds((NS - 1) * RPS, LASTR)],
                        y_sh.at[pl.ds((NS - 1) * RPS, LASTR)])

    def fill_z(r, _):
      for j in range(d // 16):
        zbuf[r, pl.ds(j * 16, 16)] = jnp.zeros((16,), jnp.float32)
      return 0
    lax.fori_loop(0, ZR, fill_z, 0)

    for k in range(RPS // ZR):
      pltpu.sync_copy(zbuf, acc.at[pl.ds(s * RPS + k * ZR, ZR)])
    plsc.subcore_barrier()

    y_src = y_sh if stage_y else y_hbm

    def gather(i, b, gs):
      pltpu.async_copy(y_src.at[src_v.at[i]], b, gs)

    def wait_gather(i, b, gs):
      pltpu.make_async_copy(y_src.at[src_v.at[i]], b, gs).wait()

    def sync_scatter(i, b):
      pltpu.sync_copy(b, acc.at[dst_v.at[i]], add=True)

    for b in range(NBUF):
      gather(b, bufs[b], gsems[b])

    def group(j, _):
      g = j * NBUF
      for b in range(NBUF):
        wait_gather(g + b, bufs[b], gsems[b])
        sync_scatter(g + b, bufs[b])
        gather(g + NBUF + b, bufs[b], gsems[b])
      return 0
    lax.fori_loop(0, NGRP - 1, group, 0)

    last = (NGRP - 1) * NBUF
    for b in range(NBUF):
      wait_gather(last + b, bufs[b], gsems[b])
      sync_scatter(last + b, bufs[b])

    plsc.subcore_barrier()
    _drain(acc, out_hbm, c, s)

  return agg_kernel


_deg_call = _make_deg()
_agg16_call = _make_agg(D_H, stage_y=True)
_agg64_call = _make_agg(D_OUT, stage_y=False)


BN = 2000
BP16 = BN * D_H // 128


def _layer1_body(x_ref, w1b_ref, dg_ref, y1_ref, dinv_ref):
  deg = dg_ref[0, 0] + dg_ref[1, 0] + 1.0
  dinv = lax.rsqrt(deg)
  xw = jnp.dot(x_ref[0], w1b_ref[...], preferred_element_type=jnp.float32)
  y1_ref[0] = xw * dinv
  dinv_ref[0] = dinv


def _layer1_call(x_r8, w1b, degp_p):
  return pl.pallas_call(
      _layer1_body,
      grid=(N // BN,),
      in_specs=[
          pl.BlockSpec((1, BP16, 8 * D_IN), lambda i: (i, 0, 0)),
          pl.BlockSpec((8 * D_IN, 128), lambda i: (0, 0)),
          pl.BlockSpec((NC, 1, BP16, 128), lambda i: (0, i, 0, 0)),
      ],
      out_specs=[
          pl.BlockSpec((1, BP16, 128), lambda i: (i, 0, 0)),
          pl.BlockSpec((1, BP16, 128), lambda i: (i, 0, 0)),
      ],
      out_shape=[
          jax.ShapeDtypeStruct((N // BN, BP16, 128), jnp.float32),
          jax.ShapeDtypeStruct((N // BN, BP16, 128), jnp.float32),
      ],
  )(x_r8, w1b, degp_p)


def _layer2_body(ag_ref, y1_ref, dinv_ref, b1_ref, w2b_ref, es_ref, y2_ref):
  t = ag_ref[0, 0] + ag_ref[1, 0] + y1_ref[0]
  h = jnp.maximum(dinv_ref[0] * t + b1_ref[...], 0.0)
  hw = jnp.dot(h, w2b_ref[...], preferred_element_type=jnp.float32)
  dinv64 = jnp.dot(dinv_ref[0], es_ref[...],
                   preferred_element_type=jnp.float32)
  y2_ref[0] = hw * dinv64


def _layer2_call(agg1_p, y1p, dinvp, b1t, w2b, es):
  return pl.pallas_call(
      _layer2_body,
      grid=(N // BN,),
      in_specs=[
          pl.BlockSpec((NC, 1, BP16, 128), lambda i: (0, i, 0, 0)),
          pl.BlockSpec((1, BP16, 128), lambda i: (i, 0, 0)),
          pl.BlockSpec((1, BP16, 128), lambda i: (i, 0, 0)),
          pl.BlockSpec((1, 128), lambda i: (0, 0)),
          pl.BlockSpec((128, 8 * D_OUT), lambda i: (0, 0)),
          pl.BlockSpec((128, 8 * D_OUT), lambda i: (0, 0)),
      ],
      out_specs=pl.BlockSpec((1, BP16, 8 * D_OUT), lambda i: (i, 0, 0)),
      out_shape=jax.ShapeDtypeStruct((N // BN, BP16, 8 * D_OUT), jnp.float32),
  )(agg1_p, y1p, dinvp, b1t, w2b, es)


def _out_body(ag_ref, y2_ref, dinv_ref, es_ref, b2_ref, ms_ref, o_ref):
  dinv64 = jnp.dot(dinv_ref[0], es_ref[...],
                   preferred_element_type=jnp.float32)
  o = dinv64 * (ag_ref[0, 0] + ag_ref[1, 0] + y2_ref[0]) + b2_ref[...]
  m = jnp.max(o, axis=1, keepdims=True)
  s = jnp.dot(jnp.exp(o - m), ms_ref[...], preferred_element_type=jnp.float32)
  o_ref[0] = o - (jnp.log(s) + m)


def _out_call(agg2_v, y2p, dinvp, es, b2t, ms):
  return pl.pallas_call(
      _out_body,
      grid=(N // BN,),
      in_specs=[
          pl.BlockSpec((NC, 1, BP16, 8 * D_OUT), lambda i: (0, i, 0, 0)),
          pl.BlockSpec((1, BP16, 8 * D_OUT), lambda i: (i, 0, 0)),
          pl.BlockSpec((1, BP16, 128), lambda i: (i, 0, 0)),
          pl.BlockSpec((128, 8 * D_OUT), lambda i: (0, 0)),
          pl.BlockSpec((1, 8 * D_OUT), lambda i: (0, 0)),
          pl.BlockSpec((8 * D_OUT, 8 * D_OUT), lambda i: (0, 0)),
      ],
      out_specs=pl.BlockSpec((1, BP16, 8 * D_OUT), lambda i: (i, 0, 0)),
      out_shape=jax.ShapeDtypeStruct((N // BN, BP16, 8 * D_OUT), jnp.float32),
  )(agg2_v, y2p, dinvp, es, b2t, ms)


def kernel(x, edge_index, W1, b1, W2, b2):
  f32 = jnp.float32
  eye8 = jnp.eye(8, dtype=f32)
  ei = edge_index.astype(jnp.int32).reshape(2, NW, NCHUNK, CB)

  nb = N // BN
  degp = _deg_call(ei)
  degp_p = degp.reshape(NC, nb, BP16, 128)

  x_r8 = x.reshape(nb, BP16, 8 * D_IN)
  w1b = jnp.kron(eye8, W1)
  y1p, dinvp = _layer1_call(x_r8, w1b, degp_p)

  sel = (jnp.arange(D_H) == 0).astype(f32)[:, None] * jnp.ones((1, D_OUT), f32)
  es = jnp.kron(eye8, sel)

  agg1 = _agg16_call(y1p.reshape(N, D_H), ei)
  w2b = jnp.kron(eye8, W2)
  b1t = jnp.tile(b1, 8).reshape(1, 128)
  y2p = _layer2_call(agg1.reshape(NC, nb, BP16, 128), y1p, dinvp, b1t, w2b,
                     es)

  agg2 = _agg64_call(y2p.reshape(N, D_OUT), ei)
  ms = jnp.kron(eye8, jnp.ones((D_OUT, D_OUT), f32))
  b2t = jnp.tile(b2, 8).reshape(1, 8 * D_OUT)
  outp = _out_call(agg2.reshape(NC, nb, BP16, 8 * D_OUT), y2p, dinvp, es,
                   b2t, ms)
  return outp.reshape(N, D_OUT)
